# baseline (speedup 1.0000x reference)
import functools

import jax
import jax.numpy as jnp
from jax import lax
from jax.experimental import pallas as pl
from jax.experimental.pallas import tpu as pltpu, tpu_sc as plsc

N_NODES = 100000
N_EDGES = 6400000
NUM_GRAPHS = 1000
NUM_OBS = 10
DIM_EMBED = 16

NC = 2
NS = 16
NW = NC * NS

EC = 128
E_ROWS = N_EDGES // EC
EB_BASE = E_ROWS // NW
EB_REM = E_ROWS % NW
EIB = 16
EBLK = EB_BASE // EIB
EDEPTH = 8
ELOOK = 4
ACC_ROWS = 100352
LIVE_ROWS = N_NODES // NS
ZCHUNK = 125


def _sc_edge_scatter_body(h_hbm, src_hbm, dst_hbm, out_hbm,
                          sidx_v, didx_v, rows_v, zbuf_v, acc_sh,
                          gsem, ssem, isem):
    c = lax.axis_index("c")
    s = lax.axis_index("s")
    w = c * NS + s

    def zrow(i, carry):
        zbuf_v[i, :] = jnp.zeros((16,), jnp.float32)
        return carry
    lax.fori_loop(0, ZCHUNK, zrow, 0)

    tbase = s * LIVE_ROWS

    def zchunk(i, carry):
        pltpu.sync_copy(zbuf_v, acc_sh.at[pl.ds(tbase + i * ZCHUNK, ZCHUNK)])
        return carry
    lax.fori_loop(0, LIVE_ROWS // ZCHUNK, zchunk, 0)

    plsc.subcore_barrier()

    nrows = EB_BASE + jnp.where(w < EB_REM, 1, 0)
    base = w * EB_BASE + jnp.minimum(w, EB_REM)


    def do_block(p, bk):
        if bk is not None:
            r1 = base + bk * EIB
            pltpu.async_copy(src_hbm.at[pl.ds(r1, EIB)],
                             sidx_v.at[1 - p], isem[1 - p])
            pltpu.async_copy(dst_hbm.at[pl.ds(r1, EIB)],
                             didx_v.at[1 - p], isem[1 - p])
        pltpu.make_async_copy(src_hbm.at[pl.ds(0, EIB)],
                              sidx_v.at[p], isem[p]).wait()
        pltpu.make_async_copy(dst_hbm.at[pl.ds(0, EIB)],
                              didx_v.at[p], isem[p]).wait()
        si = sidx_v.at[p]
        di = didx_v.at[p]
        gd = [None] * EDEPTH
        sd = [None] * EDEPTH
        for b in range(ELOOK):
            gd[b] = pltpu.async_copy(h_hbm.at[si.at[b]],
                                     rows_v.at[b], gsem[b])
        for j in range(EIB):
            nb = (j + ELOOK) % EDEPTH
            if j + ELOOK < EIB:
                if sd[nb] is not None:
                    sd[nb].wait()
                gd[nb] = pltpu.async_copy(h_hbm.at[si.at[j + ELOOK]],
                                          rows_v.at[nb], gsem[nb])
            gd[j % EDEPTH].wait()
            sd[j % EDEPTH] = pltpu.async_copy(rows_v.at[j % EDEPTH],
                                              acc_sh.at[di.at[j]],
                                              ssem[j % EDEPTH], add=True)
        for b in range(EDEPTH):
            if sd[b] is not None:
                sd[b].wait()

    pltpu.async_copy(src_hbm.at[pl.ds(base, EIB)], sidx_v.at[0], isem[0])
    pltpu.async_copy(dst_hbm.at[pl.ds(base, EIB)], didx_v.at[0], isem[0])

    def edge_pair(g, carry):
        do_block(0, 2 * g + 1)
        do_block(1, 2 * g + 2)
        return carry
    lax.fori_loop(0, EBLK // 2, edge_pair, 0)
    do_block(0, None)

    def edge_row(r, carry):
        row = base + EBLK * EIB + r
        pltpu.sync_copy(src_hbm.at[row], sidx_v.at[0, 0])
        pltpu.sync_copy(dst_hbm.at[row], didx_v.at[0, 0])
        pltpu.async_copy(h_hbm.at[sidx_v.at[0, 0]], rows_v.at[0],
                         gsem[0]).wait()
        pltpu.sync_copy(rows_v.at[0], acc_sh.at[didx_v.at[0, 0]], add=True)
        return carry
    lax.fori_loop(0, nrows - EBLK * EIB, edge_row, 0)

    plsc.subcore_barrier()

    def out_chunk(i, carry):
        pltpu.sync_copy(acc_sh.at[pl.ds(tbase + i * ZCHUNK, ZCHUNK)],
                        out_hbm.at[c, pl.ds(tbase + i * ZCHUNK, ZCHUNK)])
        return carry
    lax.fori_loop(0, LIVE_ROWS // ZCHUNK, out_chunk, 0)


def _sc_edge_scatter(h, src2, dst2):
    mesh = plsc.VectorSubcoreMesh(core_axis_name="c", subcore_axis_name="s")
    f = pl.kernel(
        _sc_edge_scatter_body,
        out_type=jax.ShapeDtypeStruct((NC, N_NODES, DIM_EMBED), jnp.float32),
        mesh=mesh,
        scratch_types=[
            pltpu.VMEM((2, EIB, EC), jnp.int32),
            pltpu.VMEM((2, EIB, EC), jnp.int32),
            pltpu.VMEM((EDEPTH, EC, DIM_EMBED), jnp.float32),
            pltpu.VMEM((ZCHUNK, DIM_EMBED), jnp.float32),
            pltpu.VMEM_SHARED((ACC_ROWS, DIM_EMBED), jnp.float32),
            [pltpu.SemaphoreType.DMA] * EDEPTH,
            [pltpu.SemaphoreType.DMA] * EDEPTH,
            [pltpu.SemaphoreType.DMA] * 2,
        ],
        compiler_params=pltpu.CompilerParams(use_tc_tiling_on_sc=False),
    )
    return f(h, src2, dst2)


RB = 4000
NB = N_NODES // RB


def _mm(x, w, b):
    return jnp.dot(x, w, preferred_element_type=jnp.float32) + b


def _tc_prep_body(x_ref, w0, b0, w1, b1, w2, b2, o_ref):
    h = jnp.maximum(_mm(x_ref[...], w0[...], b0[...]), 0.0)
    h = jnp.maximum(_mm(h, w1[...], b1[...]), 0.0)
    o_ref[...] = _mm(h, w2[...], b2[...])


def _tc_prep(x, prep):
    (w0, b0), (w1, b1), (w2, b2) = prep
    wspecs = [pl.BlockSpec(a.shape, lambda i: (0,) * a.ndim)
              for p in prep for a in (p[0], p[1].reshape(1, -1))]
    wargs = [a for p in prep for a in (p[0], p[1].reshape(1, -1))]
    return pl.pallas_call(
        _tc_prep_body,
        grid=(NB,),
        in_specs=[pl.BlockSpec((RB, 6), lambda i: (i, 0))] + wspecs,
        out_specs=pl.BlockSpec((RB, DIM_EMBED), lambda i: (i, 0)),
        out_shape=jax.ShapeDtypeStruct((N_NODES, DIM_EMBED), jnp.float32),
    )(x, *wargs)


SEG_PAD = 1024


def _tc_fused_body(x_ref, p_ref, b_ref, ip_ref,
                   aw0, ab0, aw1, ab1, aw2, ab2,
                   pw0, pb0, pw1, pb1, pw2, pb2,
                   nw0, nb0, nw1, nb1, nw2, nb2,
                   dw0, db0, dw1, db1, dw2, db2,
                   ne_ref, de_ref, z_ref, dacc):
    i = pl.program_id(0)

    @pl.when(i == 0)
    def _():
        dacc[...] = jnp.zeros((SEG_PAD, DIM_EMBED), jnp.float32)

    x = x_ref[...]
    xa = p_ref[0] + p_ref[1]
    a = jnp.maximum(_mm(xa, aw0[...], ab0[...]), 0.0)
    a = jnp.maximum(_mm(a, aw1[...], ab1[...]), 0.0)
    a = _mm(a, aw2[...], ab2[...])
    t = jnp.maximum(jnp.dot(x, pw0[0:6, :], preferred_element_type=jnp.float32)
                    + jnp.dot(a, pw0[6:22, :],
                              preferred_element_type=jnp.float32)
                    + pb0[...], 0.0)
    t = jnp.maximum(_mm(t, pw1[...], pb1[...]), 0.0)
    ne = _mm(t, pw2[...], pb2[...])
    ne_ref[...] = ne
    u = jnp.maximum(jnp.dot(x, nw0[0:6, :], preferred_element_type=jnp.float32)
                    + jnp.dot(ne, nw0[6:22, :],
                              preferred_element_type=jnp.float32)
                    + nb0[...], 0.0)
    u = jnp.maximum(_mm(u, nw1[...], nb1[...]), 0.0)
    mg = _mm(u, nw2[...], nb2[...])

    seg = lax.broadcasted_iota(jnp.int32, (RB, SEG_PAD), 1)
    oh = (seg == b_ref[...]).astype(jnp.float32)
    dacc[...] += lax.dot_general(oh, mg, (((0,), (0,)), ((), ())),
                                 preferred_element_type=jnp.float32)

    @pl.when(i == NB - 1)
    def _():
        de = dacc[0:NUM_GRAPHS, :]
        de_ref[...] = de
        dp = jnp.maximum(_mm(de, dw0[...], db0[...]), 0.0)
        dp = jnp.maximum(_mm(dp, dw1[...], db1[...]), 0.0)
        dp = _mm(dp, dw2[...], db2[...])
        lower = ip_ref[0:NUM_OBS, :]
        upper = ip_ref[1:NUM_OBS + 1, :]
        g = lax.broadcasted_iota(jnp.int32, (NUM_OBS, NUM_GRAPHS), 1)
        ohp = jnp.logical_and(g >= lower, g < upper).astype(jnp.float32)
        counts = (upper - lower).astype(jnp.float32)
        z_ref[...] = jnp.dot(ohp, dp, preferred_element_type=jnp.float32) \
            / jnp.maximum(counts, 1.0)


def _tc_fused(x, parts, batch, obs_indptr, params):
    wargs = [a for mlpp in (params["agg"], params["proc"], params["node"],
                            params["dag"])
             for p in mlpp for a in (p[0], p[1].reshape(1, -1))]
    wspecs = [pl.BlockSpec(a.shape, lambda i: (0,) * a.ndim) for a in wargs]
    return pl.pallas_call(
        _tc_fused_body,
        grid=(NB,),
        in_specs=[pl.BlockSpec((RB, 6), lambda i: (i, 0)),
                  pl.BlockSpec((NC, RB, DIM_EMBED), lambda i: (0, i, 0)),
                  pl.BlockSpec((RB, 1), lambda i: (i, 0)),
                  pl.BlockSpec((NUM_OBS + 1, 1), lambda i: (0, 0))]
                 + wspecs,
        out_specs=[pl.BlockSpec((RB, DIM_EMBED), lambda i: (i, 0)),
                   pl.BlockSpec((NUM_GRAPHS, DIM_EMBED), lambda i: (0, 0)),
                   pl.BlockSpec((NUM_OBS, DIM_EMBED), lambda i: (0, 0))],
        out_shape=[jax.ShapeDtypeStruct((N_NODES, DIM_EMBED), jnp.float32),
                   jax.ShapeDtypeStruct((NUM_GRAPHS, DIM_EMBED), jnp.float32),
                   jax.ShapeDtypeStruct((NUM_OBS, DIM_EMBED), jnp.float32)],
        scratch_shapes=[pltpu.VMEM((SEG_PAD, DIM_EMBED), jnp.float32)],
    )(x, parts, batch.reshape(N_NODES, 1),
      obs_indptr.reshape(NUM_OBS + 1, 1), *wargs)


def kernel(x, edge_index, batch, obs_indptr, params):
    h = _tc_prep(x, params["prep"])
    src2 = edge_index[0].reshape(E_ROWS, EC)
    dst2 = edge_index[1].reshape(E_ROWS, EC)
    parts = _sc_edge_scatter(h, src2, dst2)
    node_emb, dag_emb, z = _tc_fused(x, parts, batch, obs_indptr, params)
    return node_emb, dag_emb, z


# speedup vs baseline: 1.0629x; 1.0629x over previous
import jax
import jax.numpy as jnp
from jax import lax
from jax.experimental import pallas as pl
from jax.experimental.pallas import tpu as pltpu, tpu_sc as plsc

N_NODES = 100000
N_EDGES = 6400000
NUM_GRAPHS = 1000
NUM_OBS = 10
DIM_EMBED = 16

NC = 2
NS = 16
NW = NC * NS

EC = 128
E_ROWS = N_EDGES // EC
EB_BASE = E_ROWS // NW
EB_REM = E_ROWS % NW
EIB = 16
EBLK = EB_BASE // EIB
EDEPTH = 8
ELOOK = 4
ACC_ROWS = 100352
LIVE_ROWS = N_NODES // NS
ZCHUNK = 125


def _sc_edge_scatter_body(h_hbm, e_hbm, out_hbm,
                          sidx_v, didx_v, rows_v, zbuf_v, acc_sh,
                          gsem, ssem, isem):
    c = lax.axis_index("c")
    s = lax.axis_index("s")
    w = c * NS + s
    src_hbm = e_hbm.at[0]
    dst_hbm = e_hbm.at[1]

    def zrow(i, carry):
        zbuf_v[i, :] = jnp.zeros((16,), jnp.float32)
        return carry
    lax.fori_loop(0, ZCHUNK, zrow, 0)

    tbase = s * LIVE_ROWS

    def zchunk(i, carry):
        pltpu.sync_copy(zbuf_v, acc_sh.at[pl.ds(tbase + i * ZCHUNK, ZCHUNK)])
        return carry
    lax.fori_loop(0, LIVE_ROWS // ZCHUNK, zchunk, 0)

    plsc.subcore_barrier()

    nrows = EB_BASE + jnp.where(w < EB_REM, 1, 0)
    base = w * EB_BASE + jnp.minimum(w, EB_REM)


    def do_block(p, bk):
        if bk is not None:
            r1 = base + bk * EIB
            pltpu.async_copy(src_hbm.at[pl.ds(r1, EIB)],
                             sidx_v.at[1 - p], isem[1 - p])
            pltpu.async_copy(dst_hbm.at[pl.ds(r1, EIB)],
                             didx_v.at[1 - p], isem[1 - p])
        pltpu.make_async_copy(src_hbm.at[pl.ds(0, EIB)],
                              sidx_v.at[p], isem[p]).wait()
        pltpu.make_async_copy(dst_hbm.at[pl.ds(0, EIB)],
                              didx_v.at[p], isem[p]).wait()
        si = sidx_v.at[p]
        di = didx_v.at[p]
        gd = [None] * EDEPTH
        sd = [None] * EDEPTH
        for b in range(ELOOK):
            gd[b] = pltpu.async_copy(h_hbm.at[si.at[b]],
                                     rows_v.at[b], gsem[b])
        for j in range(EIB):
            nb = (j + ELOOK) % EDEPTH
            if j + ELOOK < EIB:
                if sd[nb] is not None:
                    sd[nb].wait()
                gd[nb] = pltpu.async_copy(h_hbm.at[si.at[j + ELOOK]],
                                          rows_v.at[nb], gsem[nb])
            gd[j % EDEPTH].wait()
            sd[j % EDEPTH] = pltpu.async_copy(rows_v.at[j % EDEPTH],
                                              acc_sh.at[di.at[j]],
                                              ssem[j % EDEPTH], add=True)
        for b in range(EDEPTH):
            if sd[b] is not None:
                sd[b].wait()

    pltpu.async_copy(src_hbm.at[pl.ds(base, EIB)], sidx_v.at[0], isem[0])
    pltpu.async_copy(dst_hbm.at[pl.ds(base, EIB)], didx_v.at[0], isem[0])

    def edge_pair(g, carry):
        do_block(0, 2 * g + 1)
        do_block(1, 2 * g + 2)
        return carry
    lax.fori_loop(0, EBLK // 2, edge_pair, 0)
    do_block(0, None)

    def edge_row(r, carry):
        row = base + EBLK * EIB + r
        pltpu.sync_copy(src_hbm.at[row], sidx_v.at[0, 0])
        pltpu.sync_copy(dst_hbm.at[row], didx_v.at[0, 0])
        pltpu.async_copy(h_hbm.at[sidx_v.at[0, 0]], rows_v.at[0],
                         gsem[0]).wait()
        pltpu.sync_copy(rows_v.at[0], acc_sh.at[didx_v.at[0, 0]], add=True)
        return carry
    lax.fori_loop(0, nrows - EBLK * EIB, edge_row, 0)

    plsc.subcore_barrier()

    def out_chunk(i, carry):
        pltpu.sync_copy(acc_sh.at[pl.ds(tbase + i * ZCHUNK, ZCHUNK)],
                        out_hbm.at[c, pl.ds(tbase + i * ZCHUNK, ZCHUNK)])
        return carry
    lax.fori_loop(0, LIVE_ROWS // ZCHUNK, out_chunk, 0)


def _sc_edge_scatter(h, edges3):
    mesh = plsc.VectorSubcoreMesh(core_axis_name="c", subcore_axis_name="s")
    f = pl.kernel(
        _sc_edge_scatter_body,
        out_type=jax.ShapeDtypeStruct((NC, N_NODES, DIM_EMBED), jnp.float32),
        mesh=mesh,
        scratch_types=[
            pltpu.VMEM((2, EIB, EC), jnp.int32),
            pltpu.VMEM((2, EIB, EC), jnp.int32),
            pltpu.VMEM((EDEPTH, EC, DIM_EMBED), jnp.float32),
            pltpu.VMEM((ZCHUNK, DIM_EMBED), jnp.float32),
            pltpu.VMEM_SHARED((ACC_ROWS, DIM_EMBED), jnp.float32),
            [pltpu.SemaphoreType.DMA] * EDEPTH,
            [pltpu.SemaphoreType.DMA] * EDEPTH,
            [pltpu.SemaphoreType.DMA] * 2,
        ],
        compiler_params=pltpu.CompilerParams(use_tc_tiling_on_sc=False),
    )
    return f(h, edges3)


DC = 80
D_ROWS = N_NODES // DC
DB_BASE = D_ROWS // NW
DB_REM = D_ROWS % NW
DACC_ROWS = 1024


def _sc_dag_scatter_body(m_hbm, b_hbm, out_hbm, bidx_v, mrows_v, zbuf_v,
                         acc_sh, lsem, ssem):
    c = lax.axis_index("c")
    s = lax.axis_index("s")
    w = c * NS + s

    def zrow(i, carry):
        zbuf_v[i, :] = jnp.zeros((16,), jnp.float32)
        return carry
    lax.fori_loop(0, DACC_ROWS // NS, zrow, 0)
    pltpu.sync_copy(zbuf_v, acc_sh.at[pl.ds(s * (DACC_ROWS // NS),
                                            DACC_ROWS // NS)])
    plsc.subcore_barrier()

    base = w * DB_BASE + jnp.minimum(w, DB_REM)

    def issue_loads(r, b):
        pltpu.async_copy(b_hbm.at[base + r], bidx_v.at[b], lsem[b])
        pltpu.async_copy(m_hbm.at[base + r], mrows_v.at[b], lsem[b])

    sd = [None] * 4
    for b in range(3):
        issue_loads(b, b)
    for j in range(DB_BASE):
        nb = (j + 3) % 4
        if j + 3 < DB_BASE:
            if sd[nb] is not None:
                sd[nb].wait()
            issue_loads(j + 3, nb)
        b = j % 4
        pltpu.make_async_copy(b_hbm.at[base], bidx_v.at[b], lsem[b]).wait()
        pltpu.make_async_copy(m_hbm.at[base], mrows_v.at[b], lsem[b]).wait()
        sd[b] = pltpu.async_copy(mrows_v.at[b], acc_sh.at[bidx_v.at[b]],
                                 ssem[b], add=True)
    for b in range(4):
        if sd[b] is not None:
            sd[b].wait()

    @pl.when(w < DB_REM)
    def _():
        row = base + DB_BASE
        pltpu.sync_copy(b_hbm.at[row], bidx_v.at[0])
        pltpu.sync_copy(m_hbm.at[row], mrows_v.at[0])
        pltpu.sync_copy(mrows_v.at[0], acc_sh.at[bidx_v.at[0]], add=True)

    plsc.subcore_barrier()

    @pl.when(s == 0)
    def _():
        pltpu.sync_copy(acc_sh.at[pl.ds(0, NUM_GRAPHS)], out_hbm.at[c])


def _sc_dag_scatter(merged3, batch2):
    mesh = plsc.VectorSubcoreMesh(core_axis_name="c", subcore_axis_name="s")
    f = pl.kernel(
        _sc_dag_scatter_body,
        out_type=jax.ShapeDtypeStruct((NC, NUM_GRAPHS, DIM_EMBED),
                                      jnp.float32),
        mesh=mesh,
        scratch_types=[
            pltpu.VMEM((4, DC), jnp.int32),
            pltpu.VMEM((4, DC, DIM_EMBED), jnp.float32),
            pltpu.VMEM((DACC_ROWS // NS, DIM_EMBED), jnp.float32),
            pltpu.VMEM_SHARED((DACC_ROWS, DIM_EMBED), jnp.float32),
            [pltpu.SemaphoreType.DMA] * 4,
            [pltpu.SemaphoreType.DMA] * 4,
        ],
        compiler_params=pltpu.CompilerParams(use_tc_tiling_on_sc=False),
    )
    return f(merged3, batch2)


RB = 4000
NB = N_NODES // RB


def _mm(x, w, b):
    return jnp.dot(x, w, preferred_element_type=jnp.float32) + b


def _tc_prep_body(x_ref, w0, b0, w1, b1, w2, b2, o_ref):
    h = jnp.maximum(_mm(x_ref[...], w0[...], b0[...]), 0.0)
    h = jnp.maximum(_mm(h, w1[...], b1[...]), 0.0)
    o_ref[...] = _mm(h, w2[...], b2[...])


def _tc_prep(x, prep):
    wspecs = [pl.BlockSpec(a.shape, lambda i: (0,) * a.ndim)
              for p in prep for a in (p[0], p[1].reshape(1, -1))]
    wargs = [a for p in prep for a in (p[0], p[1].reshape(1, -1))]
    return pl.pallas_call(
        _tc_prep_body,
        grid=(NB,),
        in_specs=[pl.BlockSpec((RB, 6), lambda i: (i, 0))] + wspecs,
        out_specs=pl.BlockSpec((RB, DIM_EMBED), lambda i: (i, 0)),
        out_shape=jax.ShapeDtypeStruct((N_NODES, DIM_EMBED), jnp.float32),
    )(x, *wargs)


def _tc_mid_body(x_ref, p_ref,
                 aw0, ab0, aw1, ab1, aw2, ab2,
                 pw0, pb0, pw1, pb1, pw2, pb2,
                 nw0, nb0, nw1, nb1, nw2, nb2,
                 ne_ref, mg_ref):
    x = x_ref[...]
    xa = p_ref[0] + p_ref[1]
    a = jnp.maximum(_mm(xa, aw0[...], ab0[...]), 0.0)
    a = jnp.maximum(_mm(a, aw1[...], ab1[...]), 0.0)
    a = _mm(a, aw2[...], ab2[...])
    t = jnp.maximum(jnp.dot(x, pw0[0:6, :], preferred_element_type=jnp.float32)
                    + jnp.dot(a, pw0[6:22, :],
                              preferred_element_type=jnp.float32)
                    + pb0[...], 0.0)
    t = jnp.maximum(_mm(t, pw1[...], pb1[...]), 0.0)
    ne = _mm(t, pw2[...], pb2[...])
    ne_ref[...] = ne
    u = jnp.maximum(jnp.dot(x, nw0[0:6, :], preferred_element_type=jnp.float32)
                    + jnp.dot(ne, nw0[6:22, :],
                              preferred_element_type=jnp.float32)
                    + nb0[...], 0.0)
    u = jnp.maximum(_mm(u, nw1[...], nb1[...]), 0.0)
    mg_ref[...] = _mm(u, nw2[...], nb2[...])


def _tc_mid(x, parts, agg, proc, node):
    wargs = [a for mlpp in (agg, proc, node)
             for p in mlpp for a in (p[0], p[1].reshape(1, -1))]
    wspecs = [pl.BlockSpec(a.shape, lambda i: (0,) * a.ndim) for a in wargs]
    return pl.pallas_call(
        _tc_mid_body,
        grid=(NB,),
        in_specs=[pl.BlockSpec((RB, 6), lambda i: (i, 0)),
                  pl.BlockSpec((NC, RB, DIM_EMBED), lambda i: (0, i, 0))]
                 + wspecs,
        out_specs=[pl.BlockSpec((RB, DIM_EMBED), lambda i: (i, 0)),
                   pl.BlockSpec((RB, DIM_EMBED), lambda i: (i, 0))],
        out_shape=[jax.ShapeDtypeStruct((N_NODES, DIM_EMBED), jnp.float32),
                   jax.ShapeDtypeStruct((N_NODES, DIM_EMBED), jnp.float32)],
    )(x, parts, *wargs)


def _tc_tail_body(d_ref, ip_ref,
                  dw0, db0, dw1, db1, dw2, db2,
                  de_ref, z_ref):
    de = d_ref[0] + d_ref[1]
    de_ref[...] = de
    dp = jnp.maximum(_mm(de, dw0[...], db0[...]), 0.0)
    dp = jnp.maximum(_mm(dp, dw1[...], db1[...]), 0.0)
    dp = _mm(dp, dw2[...], db2[...])
    lower = ip_ref[0:NUM_OBS, :]
    upper = ip_ref[1:NUM_OBS + 1, :]
    g = lax.broadcasted_iota(jnp.int32, (NUM_OBS, NUM_GRAPHS), 1)
    oh = jnp.logical_and(g >= lower, g < upper).astype(jnp.float32)
    counts = (upper - lower).astype(jnp.float32)
    z_ref[...] = jnp.dot(oh, dp, preferred_element_type=jnp.float32) \
        / jnp.maximum(counts, 1.0)


def _tc_tail(dparts, obs_indptr, dag):
    wargs = [a for p in dag for a in (p[0], p[1].reshape(1, -1))]
    return pl.pallas_call(
        _tc_tail_body,
        out_shape=[jax.ShapeDtypeStruct((NUM_GRAPHS, DIM_EMBED), jnp.float32),
                   jax.ShapeDtypeStruct((NUM_OBS, DIM_EMBED), jnp.float32)],
    )(dparts, obs_indptr.reshape(NUM_OBS + 1, 1), *wargs)


def kernel(x, edge_index, batch, obs_indptr, params):
    h = _tc_prep(x, params["prep"])
    edges3 = edge_index.reshape(2, E_ROWS, EC)
    parts = _sc_edge_scatter(h, edges3)
    node_emb, merged = _tc_mid(x, parts, params["agg"], params["proc"],
                               params["node"])
    merged3 = merged.reshape(D_ROWS, DC, DIM_EMBED)
    batch2 = batch.reshape(D_ROWS, DC)
    dparts = _sc_dag_scatter(merged3, batch2)
    dag_emb, z = _tc_tail(dparts, obs_indptr, params["dag"])
    return node_emb, dag_emb, z
